# baseline (speedup 1.0000x reference)
import functools

import jax
import jax.numpy as jnp
from jax import lax
from jax.experimental import pallas as pl
from jax.experimental.pallas import tpu as pltpu
from jax.experimental.pallas import tpu_sc as plsc

_CH = 64
_SB = 4608
_IMAX = 2147483647


def _rotl(x, d):
    return jnp.left_shift(x, jnp.uint32(d)) | jnp.right_shift(x, jnp.uint32(32 - d))


def _threefry_bits(k0, k1, x1_init, shape):
    ks2 = k0 ^ k1 ^ jnp.uint32(0x1BD11BDA)
    x0 = jnp.broadcast_to(k0, shape)
    x1 = jnp.broadcast_to(x1_init, shape)
    rots = ((13, 15, 26, 6), (17, 29, 16, 24))
    ksv = (k0, k1, ks2)
    inj1 = tuple(ksv[(r + 1) % 3] for r in range(5))
    inj2 = tuple(ksv[(r + 2) % 3] + jnp.uint32(r + 1) for r in range(5))
    for r in range(5):
        for d in rots[r % 2]:
            x0 = x0 + x1
            x1 = _rotl(x1, d)
            x1 = x1 ^ x0
        x0 = x0 + inj1[r]
        x1 = x1 + inj2[r]
    return x0 ^ x1


def _gumbel_from_bits(bits):
    mant = jnp.right_shift(bits, jnp.uint32(9)) | jnp.uint32(0x3F800000)
    u = jax.lax.bitcast_convert_type(mant, jnp.float32) - jnp.float32(1.0)
    g = -jnp.log(u + jnp.float32(1e-20))
    return -jnp.log(g + jnp.float32(1e-20))


_SC_CW = 16


def _sc_bits_body(k0_hbm, k1_hbm, out_hbm, kv0_buf, kv1_buf, buf,
                  *, col0, npw, rows):
    nc = 2
    nrg = rows // 16
    w = lax.axis_index("s") * nc + lax.axis_index("c")
    base = w * npw
    pltpu.sync_copy(k0_hbm, kv0_buf)
    pltpu.sync_copy(k1_hbm, kv1_buf)
    n_full = npw // _SC_CW
    n_tail = npw - n_full * _SC_CW

    def one_col(col, jj):
        cu = (col0 + col).astype(jnp.uint32)
        for rg in range(nrg):
            kv0 = kv0_buf[rg, :]
            kv1 = kv1_buf[rg, :]
            x1 = jnp.broadcast_to(cu, (16,)) + kv1
            buf[jj, pl.ds(16 * rg, 16)] = _threefry_bits(kv0, kv1, x1, (16,))

    def chunk(ch, _):
        def col_body(jj, _):
            one_col(base + ch * _SC_CW + jj, jj)
            return 0
        lax.fori_loop(0, _SC_CW, col_body, 0)
        pltpu.sync_copy(buf, out_hbm.at[pl.ds(base + ch * _SC_CW, _SC_CW), :])
        return 0

    lax.fori_loop(0, n_full, chunk, 0)
    if n_tail:
        def tail_body(jj, _):
            one_col(base + n_full * _SC_CW + jj, jj)
            return 0
        lax.fori_loop(0, n_tail, tail_body, 0)
        pltpu.sync_copy(buf.at[pl.ds(0, n_tail)],
                        out_hbm.at[pl.ds(base + n_full * _SC_CW, n_tail), :])


def _sc_bits(k0m, k1m, col0, s, rows):
    mesh = plsc.VectorSubcoreMesh(core_axis_name="c", subcore_axis_name="s")
    fn = functools.partial(
        pl.kernel,
        mesh=mesh,
        out_type=jax.ShapeDtypeStruct((s, rows), jnp.uint32),
        scratch_types=[
            pltpu.VMEM((rows // 16, 16), jnp.uint32),
            pltpu.VMEM((rows // 16, 16), jnp.uint32),
            pltpu.VMEM((_SC_CW, rows), jnp.uint32),
        ],
    )(functools.partial(_sc_bits_body, col0=col0, npw=s // 32, rows=rows))
    return fn(k0m, k1m)


def _tc_main_body(lt_ref, k0_ref, k1_ref, st_ref, nz_ref,
                  bvp_ref, bip_ref, bv_ref, bc_ref, *, gm, rows):
    v = pl.program_id(0)

    @pl.when(v == 0)
    def _():
        bv_ref[...] = jnp.full((_CH, rows), -jnp.inf, jnp.float32)
        bc_ref[...] = jnp.zeros((_CH, rows), jnp.int32)

    k0 = k0_ref[...]
    k1 = k1_ref[...]
    st = st_ref[...]
    nz = nz_ref[...]
    subl = jax.lax.broadcasted_iota(jnp.int32, (_CH, rows), 0)
    subl_u = subl.astype(jnp.uint32)

    nch = _SB // _CH
    bv_acc = bv_ref[...]
    bc_acc = bc_ref[...]
    for c in range(nch):
        k1c = k1 + jnp.uint32(c * _CH) + (v * _SB).astype(jnp.uint32)
        bits = _threefry_bits(k0, k1, subl_u + k1c, (_CH, rows))
        noise = _gumbel_from_bits(bits)
        scaled = lt_ref[pl.ds(c * _CH, _CH), :] / st
        pert = scaled + noise * nz
        take = pert > bv_acc
        bv_acc = jnp.where(take, pert, bv_acc)
        bc_acc = jnp.where(take, v * nch + c, bc_acc)
    bv_ref[...] = bv_acc
    bc_ref[...] = bc_acc

    @pl.when(v == gm - 1)
    def _():
        fin_col = bc_acc * _CH + subl
        mx = jnp.max(bv_acc, axis=0, keepdims=True)
        idx = jnp.min(jnp.where(bv_acc == mx, fin_col, _IMAX),
                      axis=0, keepdims=True)
        bvp_ref[...] = jnp.broadcast_to(mx, (8, rows))
        bip_ref[...] = jnp.broadcast_to(idx, (8, rows))


def _tc_tail_body(bits_ref, lt_ref, st_ref, nz_ref, bvp_ref, bip_ref,
                  out_ref, bv_ref, bi_ref, *, nt, m, vocab, rows):
    v = pl.program_id(0)

    @pl.when(v == 0)
    def _():
        bv_ref[...] = jnp.full((_CH, rows), -jnp.inf, jnp.float32)
        bi_ref[...] = jnp.full((_CH, rows), _IMAX, jnp.int32)

    st = st_ref[...]
    nz = nz_ref[...]
    subl = jax.lax.broadcasted_iota(jnp.int32, (_CH, rows), 0)

    bv_acc = bv_ref[...]
    bi_acc = bi_ref[...]
    for c in range(_SB // _CH):
        cols = subl + (m + v * _SB + c * _CH)
        noise = _gumbel_from_bits(bits_ref[pl.ds(c * _CH, _CH), :])
        scaled = lt_ref[pl.ds(c * _CH, _CH), :] / st
        pert = scaled + noise * nz
        pert = jnp.where(cols < vocab, pert, -jnp.inf)
        take = pert > bv_acc
        bv_acc = jnp.where(take, pert, bv_acc)
        bi_acc = jnp.where(take, cols, bi_acc)
    bv_ref[...] = bv_acc
    bi_ref[...] = bi_acc

    @pl.when(v == nt - 1)
    def _():
        mx = jnp.max(bv_acc, axis=0, keepdims=True)
        idx = jnp.min(jnp.where(bv_acc == mx, bi_acc, _IMAX),
                      axis=0, keepdims=True)
        bvp = bvp_ref[0:1, :]
        bip = bip_ref[0:1, :]
        take = (mx > bvp) | ((mx == bvp) & (idx < bip))
        out = jnp.where(take, idx, bip)
        out_ref[...] = jnp.broadcast_to(out, (8, rows))


def kernel(logits, temperature, seed, pos, apply_temperature):
    rows, vocab = logits.shape
    if logits.dtype != jnp.float32:
        logits = logits.astype(jnp.float32)
    lt = logits.T

    m = (int(vocab * 0.7373) // _SB) * _SB
    s = vocab - m

    kd = jax.vmap(
        lambda sd, p: jax.random.key_data(jax.random.fold_in(jax.random.key(sd), p))
    )(seed, pos)
    k0 = kd[:, 0]
    k1 = kd[:, 1]

    at = jnp.asarray(apply_temperature)
    safe_t = jnp.where(temperature == 0.0, jnp.float32(1.0), temperature)
    st_eff = jnp.where(at != 0, safe_t, jnp.float32(1.0))[None, :]
    nz = (temperature != 0.0).astype(jnp.float32)[None, :]

    s_pad = ((s + 255) // 256) * 256
    bits = _sc_bits(k0.reshape(rows // 16, 16), k1.reshape(rows // 16, 16),
                    m, s_pad, rows)

    rowv = pl.BlockSpec((1, rows), lambda v: (0, 0))
    prt = pl.BlockSpec((8, rows), lambda v: (0, 0))
    gm = m // _SB
    bvp, bip = pl.pallas_call(
        functools.partial(_tc_main_body, gm=gm, rows=rows),
        grid=(gm,),
        in_specs=[
            pl.BlockSpec((_SB, rows), lambda v: (v, 0)),
            rowv, rowv, rowv, rowv,
        ],
        out_specs=[prt, prt],
        out_shape=[
            jax.ShapeDtypeStruct((8, rows), jnp.float32),
            jax.ShapeDtypeStruct((8, rows), jnp.int32),
        ],
        scratch_shapes=[
            pltpu.VMEM((_CH, rows), jnp.float32),
            pltpu.VMEM((_CH, rows), jnp.int32),
        ],
    )(lt, k0[None, :], k1[None, :], st_eff, nz)

    nt = pl.cdiv(s_pad, _SB)
    off = m // _SB
    out = pl.pallas_call(
        functools.partial(_tc_tail_body, nt=nt, m=m, vocab=vocab, rows=rows),
        grid=(nt,),
        in_specs=[
            pl.BlockSpec((_SB, rows), lambda v: (v, 0)),
            pl.BlockSpec((_SB, rows), lambda v: (v + off, 0)),
            rowv, rowv, prt, prt,
        ],
        out_specs=prt,
        out_shape=jax.ShapeDtypeStruct((8, rows), jnp.int32),
        scratch_shapes=[
            pltpu.VMEM((_CH, rows), jnp.float32),
            pltpu.VMEM((_CH, rows), jnp.int32),
        ],
    )(bits, lt, st_eff, nz, bvp, bip)
    return out[0]


# speedup vs baseline: 1.0105x; 1.0105x over previous
import functools

import jax
import jax.numpy as jnp
from jax import lax
from jax.experimental import pallas as pl
from jax.experimental.pallas import tpu as pltpu
from jax.experimental.pallas import tpu_sc as plsc

_CH = 64
_SB = 4608
_IMAX = 2147483647


def _rotl(x, d):
    return jnp.left_shift(x, jnp.uint32(d)) | jnp.right_shift(x, jnp.uint32(32 - d))


def _threefry_bits(k0, k1, x1_init, shape):
    ks2 = k0 ^ k1 ^ jnp.uint32(0x1BD11BDA)
    x0 = jnp.broadcast_to(k0, shape)
    x1 = jnp.broadcast_to(x1_init, shape)
    rots = ((13, 15, 26, 6), (17, 29, 16, 24))
    ksv = (k0, k1, ks2)
    inj1 = tuple(ksv[(r + 1) % 3] for r in range(5))
    inj2 = tuple(ksv[(r + 2) % 3] + jnp.uint32(r + 1) for r in range(5))
    for r in range(5):
        for d in rots[r % 2]:
            x0 = x0 + x1
            x1 = _rotl(x1, d)
            x1 = x1 ^ x0
        x0 = x0 + inj1[r]
        x1 = x1 + inj2[r]
    return x0 ^ x1


def _gumbel_from_bits(bits):
    mant = jnp.right_shift(bits, jnp.uint32(9)) | jnp.uint32(0x3F800000)
    u = jax.lax.bitcast_convert_type(mant, jnp.float32) - jnp.float32(1.0)
    g = -jnp.log(u + jnp.float32(1e-20))
    return -jnp.log(g + jnp.float32(1e-20))


def _sc_bits_body(k0_hbm, k1_hbm, out_hbm, kv0_buf, kv1_buf, buf,
                  *, col0, npw, rows):
    nc = 2
    nrg = rows // 16
    w = lax.axis_index("s") * nc + lax.axis_index("c")
    base = w * npw
    pltpu.sync_copy(k0_hbm, kv0_buf)
    pltpu.sync_copy(k1_hbm, kv1_buf)

    def col_body(jj, _):
        cu = (col0 + base + jj).astype(jnp.uint32)
        for rg in range(nrg):
            kv0 = kv0_buf[rg, :]
            kv1 = kv1_buf[rg, :]
            x1 = jnp.broadcast_to(cu, (16,)) + kv1
            buf[jj, pl.ds(16 * rg, 16)] = _threefry_bits(kv0, kv1, x1, (16,))
        return 0

    lax.fori_loop(0, npw, col_body, 0)
    pltpu.sync_copy(buf, out_hbm.at[pl.ds(base, npw), :])


def _sc_bits(k0m, k1m, col0, s, rows):
    mesh = plsc.VectorSubcoreMesh(core_axis_name="c", subcore_axis_name="s")
    fn = functools.partial(
        pl.kernel,
        mesh=mesh,
        out_type=jax.ShapeDtypeStruct((s, rows), jnp.uint32),
        scratch_types=[
            pltpu.VMEM((rows // 16, 16), jnp.uint32),
            pltpu.VMEM((rows // 16, 16), jnp.uint32),
            pltpu.VMEM((s // 32, rows), jnp.uint32),
        ],
    )(functools.partial(_sc_bits_body, col0=col0, npw=s // 32, rows=rows))
    return fn(k0m, k1m)


def _tc_main_body(lt_ref, k0_ref, k1_ref, st_ref, nz_ref,
                  bvp_ref, bip_ref, bv_ref, bc_ref, *, gm, rows):
    v = pl.program_id(0)

    @pl.when(v == 0)
    def _():
        bv_ref[...] = jnp.full((_CH, rows), -jnp.inf, jnp.float32)
        bc_ref[...] = jnp.zeros((_CH, rows), jnp.int32)

    k0 = k0_ref[...]
    k1 = k1_ref[...]
    st = st_ref[...]
    nz = nz_ref[...]
    subl = jax.lax.broadcasted_iota(jnp.int32, (_CH, rows), 0)
    subl_u = subl.astype(jnp.uint32)

    nch = _SB // _CH
    bv_acc = bv_ref[...]
    bc_acc = bc_ref[...]
    for c in range(nch):
        k1c = k1 + jnp.uint32(c * _CH) + (v * _SB).astype(jnp.uint32)
        bits = _threefry_bits(k0, k1, subl_u + k1c, (_CH, rows))
        noise = _gumbel_from_bits(bits)
        scaled = lt_ref[pl.ds(c * _CH, _CH), :] / st
        pert = scaled + noise * nz
        take = pert > bv_acc
        bv_acc = jnp.where(take, pert, bv_acc)
        bc_acc = jnp.where(take, v * nch + c, bc_acc)
    bv_ref[...] = bv_acc
    bc_ref[...] = bc_acc

    @pl.when(v == gm - 1)
    def _():
        fin_col = bc_acc * _CH + subl
        mx = jnp.max(bv_acc, axis=0, keepdims=True)
        idx = jnp.min(jnp.where(bv_acc == mx, fin_col, _IMAX),
                      axis=0, keepdims=True)
        bvp_ref[...] = jnp.broadcast_to(mx, (8, rows))
        bip_ref[...] = jnp.broadcast_to(idx, (8, rows))


def _tc_tail_body(bits_ref, lt_ref, st_ref, nz_ref, bvp_ref, bip_ref,
                  out_ref, bv_ref, bi_ref, *, nt, m, vocab, rows):
    v = pl.program_id(0)

    @pl.when(v == 0)
    def _():
        bv_ref[...] = jnp.full((_CH, rows), -jnp.inf, jnp.float32)
        bi_ref[...] = jnp.full((_CH, rows), _IMAX, jnp.int32)

    st = st_ref[...]
    nz = nz_ref[...]
    subl = jax.lax.broadcasted_iota(jnp.int32, (_CH, rows), 0)

    bv_acc = bv_ref[...]
    bi_acc = bi_ref[...]
    for c in range(_SB // _CH):
        cols = subl + (m + v * _SB + c * _CH)
        noise = _gumbel_from_bits(bits_ref[pl.ds(c * _CH, _CH), :])
        scaled = lt_ref[pl.ds(c * _CH, _CH), :] / st
        pert = scaled + noise * nz
        pert = jnp.where(cols < vocab, pert, -jnp.inf)
        take = pert > bv_acc
        bv_acc = jnp.where(take, pert, bv_acc)
        bi_acc = jnp.where(take, cols, bi_acc)
    bv_ref[...] = bv_acc
    bi_ref[...] = bi_acc

    @pl.when(v == nt - 1)
    def _():
        mx = jnp.max(bv_acc, axis=0, keepdims=True)
        idx = jnp.min(jnp.where(bv_acc == mx, bi_acc, _IMAX),
                      axis=0, keepdims=True)
        bvp = bvp_ref[0:1, :]
        bip = bip_ref[0:1, :]
        take = (mx > bvp) | ((mx == bvp) & (idx < bip))
        out = jnp.where(take, idx, bip)
        out_ref[...] = jnp.broadcast_to(out, (8, rows))


def kernel(logits, temperature, seed, pos, apply_temperature):
    rows, vocab = logits.shape
    if logits.dtype != jnp.float32:
        logits = logits.astype(jnp.float32)
    lt = logits.T

    m = (int(vocab * 0.7373) // _SB) * _SB
    s = vocab - m

    kd = jax.vmap(
        lambda sd, p: jax.random.key_data(jax.random.fold_in(jax.random.key(sd), p))
    )(seed, pos)
    k0 = kd[:, 0]
    k1 = kd[:, 1]

    at = jnp.asarray(apply_temperature)
    safe_t = jnp.where(temperature == 0.0, jnp.float32(1.0), temperature)
    st_eff = jnp.where(at != 0, safe_t, jnp.float32(1.0))[None, :]
    nz = (temperature != 0.0).astype(jnp.float32)[None, :]

    s_pad = ((s + 255) // 256) * 256
    bits = _sc_bits(k0.reshape(rows // 16, 16), k1.reshape(rows // 16, 16),
                    m, s_pad, rows)

    rowv = pl.BlockSpec((1, rows), lambda v: (0, 0))
    prt = pl.BlockSpec((8, rows), lambda v: (0, 0))
    gm = m // _SB
    bvp, bip = pl.pallas_call(
        functools.partial(_tc_main_body, gm=gm, rows=rows),
        grid=(gm,),
        in_specs=[
            pl.BlockSpec((_SB, rows), lambda v: (v, 0)),
            rowv, rowv, rowv, rowv,
        ],
        out_specs=[prt, prt],
        out_shape=[
            jax.ShapeDtypeStruct((8, rows), jnp.float32),
            jax.ShapeDtypeStruct((8, rows), jnp.int32),
        ],
        scratch_shapes=[
            pltpu.VMEM((_CH, rows), jnp.float32),
            pltpu.VMEM((_CH, rows), jnp.int32),
        ],
    )(lt, k0[None, :], k1[None, :], st_eff, nz)

    nt = pl.cdiv(s_pad, _SB)
    off = m // _SB
    out = pl.pallas_call(
        functools.partial(_tc_tail_body, nt=nt, m=m, vocab=vocab, rows=rows),
        grid=(nt,),
        in_specs=[
            pl.BlockSpec((_SB, rows), lambda v: (v, 0)),
            pl.BlockSpec((_SB, rows), lambda v: (v + off, 0)),
            rowv, rowv, prt, prt,
        ],
        out_specs=prt,
        out_shape=jax.ShapeDtypeStruct((8, rows), jnp.int32),
        scratch_shapes=[
            pltpu.VMEM((_CH, rows), jnp.float32),
            pltpu.VMEM((_CH, rows), jnp.int32),
        ],
    )(bits, lt, st_eff, nz, bvp, bip)
    return out[0]


# speedup vs baseline: 1.7155x; 1.6976x over previous
import functools

import jax
import jax.numpy as jnp
from jax import lax
from jax.experimental import pallas as pl
from jax.experimental.pallas import tpu as pltpu
from jax.experimental.pallas import tpu_sc as plsc

_CH = 64
_SB = 4608
_IMAX = 2147483647


def _rotl(x, d):
    return jnp.left_shift(x, jnp.uint32(d)) | jnp.right_shift(x, jnp.uint32(32 - d))


def _tf_sched(k0, k1):
    ks2 = k0 ^ k1 ^ jnp.uint32(0x1BD11BDA)
    ksv = (k0, k1, ks2)
    inj1 = tuple(ksv[(r + 1) % 3] for r in range(5))
    inj2 = tuple(ksv[(r + 2) % 3] + jnp.uint32(r + 1) for r in range(5))
    return inj1, inj2


def _tf_rounds(k0, inj1, inj2, x1_init, shape):
    rots = ((13, 15, 26, 6), (17, 29, 16, 24))
    x0 = jnp.broadcast_to(k0, shape)
    x1 = jnp.broadcast_to(x1_init, shape)
    for r in range(5):
        for d in rots[r % 2]:
            x0 = x0 + x1
            x1 = _rotl(x1, d)
            x1 = x1 ^ x0
        x0 = x0 + inj1[r]
        x1 = x1 + inj2[r]
    return x0 ^ x1


def _threefry_bits(k0, k1, x1_init, shape):
    inj1, inj2 = _tf_sched(k0, k1)
    return _tf_rounds(k0, inj1, inj2, x1_init, shape)


def _gumbel_from_bits(bits):
    mant = jnp.right_shift(bits, jnp.uint32(9)) | jnp.uint32(0x3F800000)
    u = jax.lax.bitcast_convert_type(mant, jnp.float32) - jnp.float32(1.0)
    g = -jnp.log(u + jnp.float32(1e-20))
    return -jnp.log(g + jnp.float32(1e-20))


def _sc_bits_body(k0_hbm, k1_hbm, out_hbm, kv0_buf, kv1_buf, buf,
                  *, col0, npw, rows):
    nc = 2
    nrg = rows // 16
    w = lax.axis_index("s") * nc + lax.axis_index("c")
    base = w * npw
    pltpu.sync_copy(k0_hbm, kv0_buf)
    pltpu.sync_copy(k1_hbm, kv1_buf)

    for rg in range(nrg):
        kv0 = kv0_buf[rg, :]
        kv1 = kv1_buf[rg, :]
        inj1, inj2 = _tf_sched(kv0, kv1)

        def col_body(jj, _, kv0=kv0, kv1=kv1, inj1=inj1, inj2=inj2, rg=rg):
            j0 = jj * 4
            for q in range(4):
                cu = (col0 + base + j0 + q).astype(jnp.uint32)
                x1 = jnp.broadcast_to(cu, (16,)) + kv1
                buf[j0 + q, pl.ds(16 * rg, 16)] = _tf_rounds(
                    kv0, inj1, inj2, x1, (16,))
            return 0

        lax.fori_loop(0, npw // 4, col_body, 0)
    pltpu.sync_copy(buf, out_hbm.at[pl.ds(base, npw), :])


def _sc_bits(k0m, k1m, col0, s, rows):
    mesh = plsc.VectorSubcoreMesh(core_axis_name="c", subcore_axis_name="s")
    fn = functools.partial(
        pl.kernel,
        mesh=mesh,
        out_type=jax.ShapeDtypeStruct((s, rows), jnp.uint32),
        scratch_types=[
            pltpu.VMEM((rows // 16, 16), jnp.uint32),
            pltpu.VMEM((rows // 16, 16), jnp.uint32),
            pltpu.VMEM((s // 32, rows), jnp.uint32),
        ],
    )(functools.partial(_sc_bits_body, col0=col0, npw=s // 32, rows=rows))
    return fn(k0m, k1m)


def _tc_main_body(lt_ref, k0_ref, k1_ref, st_ref, nz_ref,
                  bvp_ref, bip_ref, bv_ref, bc_ref, *, gm, rows):
    v = pl.program_id(0)

    @pl.when(v == 0)
    def _():
        bv_ref[...] = jnp.full((_CH, rows), -jnp.inf, jnp.float32)
        bc_ref[...] = jnp.zeros((_CH, rows), jnp.int32)

    k0 = k0_ref[...]
    k1 = k1_ref[...]
    st = st_ref[...]
    nz = nz_ref[...]
    subl = jax.lax.broadcasted_iota(jnp.int32, (_CH, rows), 0)
    subl_u = subl.astype(jnp.uint32)

    nch = _SB // _CH
    bv_acc = bv_ref[...]
    bc_acc = bc_ref[...]
    for c in range(nch):
        k1c = k1 + jnp.uint32(c * _CH) + (v * _SB).astype(jnp.uint32)
        bits = _threefry_bits(k0, k1, subl_u + k1c, (_CH, rows))
        noise = _gumbel_from_bits(bits)
        scaled = lt_ref[pl.ds(c * _CH, _CH), :] / st
        pert = scaled + noise * nz
        take = pert > bv_acc
        bv_acc = jnp.where(take, pert, bv_acc)
        bc_acc = jnp.where(take, v * nch + c, bc_acc)
    bv_ref[...] = bv_acc
    bc_ref[...] = bc_acc

    @pl.when(v == gm - 1)
    def _():
        fin_col = bc_acc * _CH + subl
        mx = jnp.max(bv_acc, axis=0, keepdims=True)
        idx = jnp.min(jnp.where(bv_acc == mx, fin_col, _IMAX),
                      axis=0, keepdims=True)
        bvp_ref[...] = jnp.broadcast_to(mx, (8, rows))
        bip_ref[...] = jnp.broadcast_to(idx, (8, rows))


def _tc_tail_body(bits_ref, lt_ref, st_ref, nz_ref, bvp_ref, bip_ref,
                  out_ref, bv_ref, bi_ref, *, nt, m, vocab, rows):
    v = pl.program_id(0)

    @pl.when(v == 0)
    def _():
        bv_ref[...] = jnp.full((_CH, rows), -jnp.inf, jnp.float32)
        bi_ref[...] = jnp.full((_CH, rows), _IMAX, jnp.int32)

    st = st_ref[...]
    nz = nz_ref[...]
    subl = jax.lax.broadcasted_iota(jnp.int32, (_CH, rows), 0)

    bv_acc = bv_ref[...]
    bi_acc = bi_ref[...]
    for c in range(_SB // _CH):
        cols = subl + (m + v * _SB + c * _CH)
        noise = _gumbel_from_bits(bits_ref[pl.ds(c * _CH, _CH), :])
        scaled = lt_ref[pl.ds(c * _CH, _CH), :] / st
        pert = scaled + noise * nz
        pert = jnp.where(cols < vocab, pert, -jnp.inf)
        take = pert > bv_acc
        bv_acc = jnp.where(take, pert, bv_acc)
        bi_acc = jnp.where(take, cols, bi_acc)
    bv_ref[...] = bv_acc
    bi_ref[...] = bi_acc

    @pl.when(v == nt - 1)
    def _():
        mx = jnp.max(bv_acc, axis=0, keepdims=True)
        idx = jnp.min(jnp.where(bv_acc == mx, bi_acc, _IMAX),
                      axis=0, keepdims=True)
        bvp = bvp_ref[0:1, :]
        bip = bip_ref[0:1, :]
        take = (mx > bvp) | ((mx == bvp) & (idx < bip))
        out = jnp.where(take, idx, bip)
        out_ref[...] = jnp.broadcast_to(out, (8, rows))


def kernel(logits, temperature, seed, pos, apply_temperature):
    rows, vocab = logits.shape
    if logits.dtype != jnp.float32:
        logits = logits.astype(jnp.float32)
    lt = logits.T

    m = (int(vocab * 0.7373) // _SB) * _SB
    s = vocab - m

    kd = jax.vmap(
        lambda sd, p: jax.random.key_data(jax.random.fold_in(jax.random.key(sd), p))
    )(seed, pos)
    k0 = kd[:, 0]
    k1 = kd[:, 1]

    at = jnp.asarray(apply_temperature)
    safe_t = jnp.where(temperature == 0.0, jnp.float32(1.0), temperature)
    st_eff = jnp.where(at != 0, safe_t, jnp.float32(1.0))[None, :]
    nz = (temperature != 0.0).astype(jnp.float32)[None, :]

    s_pad = ((s + 255) // 256) * 256
    bits = _sc_bits(k0.reshape(rows // 16, 16), k1.reshape(rows // 16, 16),
                    m, s_pad, rows)

    rowv = pl.BlockSpec((1, rows), lambda v: (0, 0))
    prt = pl.BlockSpec((8, rows), lambda v: (0, 0))
    gm = m // _SB
    bvp, bip = pl.pallas_call(
        functools.partial(_tc_main_body, gm=gm, rows=rows),
        grid=(gm,),
        in_specs=[
            pl.BlockSpec((_SB, rows), lambda v: (v, 0)),
            rowv, rowv, rowv, rowv,
        ],
        out_specs=[prt, prt],
        out_shape=[
            jax.ShapeDtypeStruct((8, rows), jnp.float32),
            jax.ShapeDtypeStruct((8, rows), jnp.int32),
        ],
        scratch_shapes=[
            pltpu.VMEM((_CH, rows), jnp.float32),
            pltpu.VMEM((_CH, rows), jnp.int32),
        ],
    )(lt, k0[None, :], k1[None, :], st_eff, nz)

    nt = pl.cdiv(s_pad, _SB)
    off = m // _SB
    out = pl.pallas_call(
        functools.partial(_tc_tail_body, nt=nt, m=m, vocab=vocab, rows=rows),
        grid=(nt,),
        in_specs=[
            pl.BlockSpec((_SB, rows), lambda v: (v, 0)),
            pl.BlockSpec((_SB, rows), lambda v: (v + off, 0)),
            rowv, rowv, prt, prt,
        ],
        out_specs=prt,
        out_shape=jax.ShapeDtypeStruct((8, rows), jnp.int32),
        scratch_shapes=[
            pltpu.VMEM((_CH, rows), jnp.float32),
            pltpu.VMEM((_CH, rows), jnp.int32),
        ],
    )(bits, lt, st_eff, nz, bvp, bip)
    return out[0]
